# Initial kernel scaffold; baseline (speedup 1.0000x reference)
import jax
import jax.numpy as jnp
from jax.experimental import pallas as pl


def kernel(peptide_tokens, decoration, charge, node_feat, edge_index, node_graph_ids, pos_emb, charge_emb, a_emb, phos_emb, gin_W1, gin_b1, gin_W2, gin_b2, ln_gamma, ln_beta):
    raise NotImplementedError("write your pallas kernel here")


# speedup vs baseline: 5.9018x; 5.9018x over previous
import functools

import jax
import jax.numpy as jnp
from jax import lax
from jax.experimental import pallas as pl
from jax.experimental.pallas import tpu as pltpu
from jax.experimental.pallas import tpu_sc as plsc

N, L, H = 1024, 50, 128
NUM_NODES, NUM_EDGES = 20480, 81920
GIN_IN, GIN_HID = 20, 16
LN_EPS = 1e-12

_NTILES = 16
_EPT = NUM_EDGES // _NTILES
_ECH = _EPT // 128
_NPT = NUM_NODES // _NTILES
_NCH = _NPT // 128
_GPT = N // _NTILES


def _mm_body(x_ref, w_ref, o_ref):
    o_ref[...] = jnp.dot(x_ref[...], w_ref[...], preferred_element_type=jnp.float32)


def _node_matmul(node_feat, w1):
    blk = 2048
    return pl.pallas_call(
        _mm_body,
        grid=(NUM_NODES // blk,),
        in_specs=[
            pl.BlockSpec((blk, GIN_IN), lambda i: (i, 0)),
            pl.BlockSpec((GIN_IN, GIN_HID), lambda i: (0, 0)),
        ],
        out_specs=pl.BlockSpec((blk, GIN_HID), lambda i: (i, 0)),
        out_shape=jax.ShapeDtypeStruct((NUM_NODES, GIN_HID), jnp.float32),
    )(node_feat, w1)


def _gin_sc_body(p0_hbm, src_hbm, dst_hbm, gid_hbm, b1_hbm, gout_hbm,
                 src_v, dst_v, rows_v, p_v, agg_v, gid_v, g_v, b1_v,
                 agg_sh, g_sh, sem):
    c = lax.axis_index("c")
    s = lax.axis_index("s")

    def zrow(i, _):
        p_v[i, :] = jnp.zeros((16,), jnp.float32)
        return 0
    lax.fori_loop(0, _NPT, zrow, 0)
    pltpu.sync_copy(p_v, agg_sh.at[pl.ds(s * _NPT, _NPT)])
    pltpu.sync_copy(p_v.at[pl.ds(0, _GPT)], g_sh.at[pl.ds(s * _GPT, _GPT)])
    plsc.subcore_barrier()

    pltpu.sync_copy(src_hbm.at[s], src_v)
    pltpu.sync_copy(dst_hbm.at[s], dst_v)

    def echunk(j, _):
        pltpu.async_copy(p0_hbm.at[src_v.at[j]], rows_v, sem).wait()
        pltpu.sync_copy(rows_v, agg_sh.at[dst_v.at[j]], add=True)
        return 0
    lax.fori_loop(0, _ECH, echunk, 0)
    plsc.subcore_barrier()

    nbase = s * _NPT
    pltpu.sync_copy(p0_hbm.at[pl.ds(nbase, _NPT)], p_v)
    pltpu.sync_copy(agg_sh.at[pl.ds(nbase, _NPT)], agg_v)
    pltpu.sync_copy(b1_hbm, b1_v)
    b1vec = b1_v[...]

    def hrow(i, _):
        p_v[i, :] = jnp.maximum(p_v[i, :] + agg_v[i, :] + b1vec, 0.0)
        return 0
    lax.fori_loop(0, _NPT, hrow, 0)

    pltpu.sync_copy(gid_hbm.at[s], gid_v)

    def gchunk(j, _):
        pltpu.sync_copy(p_v.at[pl.ds(j * 128, 128)], g_sh.at[gid_v.at[j]], add=True)
        return 0
    lax.fori_loop(0, _NCH, gchunk, 0)
    plsc.subcore_barrier()

    pltpu.sync_copy(g_sh.at[pl.ds(s * _GPT, _GPT)], g_v)
    pltpu.sync_copy(g_v, gout_hbm.at[c, pl.ds(s * _GPT, _GPT)])


@functools.lru_cache(maxsize=1)
def _make_gin_sc():
    return functools.partial(
        pl.kernel,
        out_type=jax.ShapeDtypeStruct((2, N, GIN_HID), jnp.float32),
        mesh=plsc.VectorSubcoreMesh(core_axis_name="c", subcore_axis_name="s"),
        compiler_params=pltpu.CompilerParams(use_tc_tiling_on_sc=False),
        scratch_types=[
        pltpu.VMEM((_ECH, 128), jnp.int32),
        pltpu.VMEM((_ECH, 128), jnp.int32),
        pltpu.VMEM((128, GIN_HID), jnp.float32),
        pltpu.VMEM((_NPT, GIN_HID), jnp.float32),
        pltpu.VMEM((_NPT, GIN_HID), jnp.float32),
        pltpu.VMEM((_NCH, 128), jnp.int32),
        pltpu.VMEM((_GPT, GIN_HID), jnp.float32),
            pltpu.VMEM((GIN_HID,), jnp.float32),
            pltpu.VMEM_SHARED((NUM_NODES, GIN_HID), jnp.float32),
            pltpu.VMEM_SHARED((N, GIN_HID), jnp.float32),
            pltpu.SemaphoreType.DMA,
        ],
    )(_gin_sc_body)


_BN = 64
_BF = _BN * L


def _embed_body(tok_ref, dec_ref, chg_ref, gp_ref, w2_ref, b2_ref,
                aemb_ref, phemb_ref, chemb_ref, pos_ref, gam_ref, bet_ref,
                o_ref):
    f32 = jnp.float32
    g = gp_ref[0] + gp_ref[1]
    gly = jnp.dot(g, w2_ref[...], preferred_element_type=f32) + b2_ref[...]

    chg = chg_ref[0]
    ioc = lax.broadcasted_iota(jnp.int32, (16, _BN), 0)
    onehot_c = (ioc == jnp.broadcast_to(chg, (16, _BN))).astype(f32)
    chg_emb = lax.dot_general(onehot_c, chemb_ref[...],
                              (((0,), (0,)), ((), ())),
                              preferred_element_type=f32)

    tok = tok_ref[...]
    iot = lax.broadcasted_iota(jnp.int32, (_BF, 32), 1)
    onehot_t = (iot == jnp.broadcast_to(tok, (_BF, 32))).astype(f32)
    emb = jnp.dot(onehot_t, aemb_ref[...], preferred_element_type=f32)

    dec = dec_ref[...]
    iod = lax.broadcasted_iota(jnp.int32, (_BF, 16), 1)
    onehot_d = (iod == jnp.broadcast_to(dec, (_BF, 16))).astype(f32)
    emb = emb + jnp.dot(onehot_d, phemb_ref[...], preferred_element_type=f32)

    rn = lax.broadcasted_iota(jnp.int32, (_BF, _BN), 0) // L
    expn = (rn == lax.broadcasted_iota(jnp.int32, (_BF, _BN), 1)).astype(f32)
    mask5 = (dec == 5).astype(f32)
    per_n = chg_emb + 0.0
    emb = emb + jnp.dot(expn, per_n, preferred_element_type=f32)
    emb = emb + mask5 * jnp.dot(expn, gly, preferred_element_type=f32)

    rl = lax.broadcasted_iota(jnp.int32, (_BF, L), 0) % L
    expl = (rl == lax.broadcasted_iota(jnp.int32, (_BF, L), 1)).astype(f32)
    emb = emb + jnp.dot(expl, pos_ref[...], preferred_element_type=f32)

    mu = jnp.mean(emb, axis=1, keepdims=True)
    d = emb - mu
    var = jnp.mean(d * d, axis=1, keepdims=True)
    o_ref[...] = d * lax.rsqrt(var + LN_EPS) * gam_ref[...] + bet_ref[...]


def _embed_ln(tok_flat, dec_flat, chg3, g_part, w2, b2, aemb32, phemb16,
              chemb16, pos50, gamma2, beta2):
    nblk = N // _BN
    return pl.pallas_call(
        _embed_body,
        grid=(nblk,),
        in_specs=[
            pl.BlockSpec((_BF, 1), lambda i: (i, 0)),
            pl.BlockSpec((_BF, 1), lambda i: (i, 0)),
            pl.BlockSpec((1, 1, _BN), lambda i: (i, 0, 0)),
            pl.BlockSpec((2, _BN, GIN_HID), lambda i: (0, i, 0)),
            pl.BlockSpec((GIN_HID, H), lambda i: (0, 0)),
            pl.BlockSpec((1, H), lambda i: (0, 0)),
            pl.BlockSpec((32, H), lambda i: (0, 0)),
            pl.BlockSpec((16, H), lambda i: (0, 0)),
            pl.BlockSpec((16, H), lambda i: (0, 0)),
            pl.BlockSpec((L, H), lambda i: (0, 0)),
            pl.BlockSpec((1, H), lambda i: (0, 0)),
            pl.BlockSpec((1, H), lambda i: (0, 0)),
        ],
        out_specs=pl.BlockSpec((_BF, H), lambda i: (i, 0)),
        out_shape=jax.ShapeDtypeStruct((N * L, H), jnp.float32),
    )(tok_flat, dec_flat, chg3, g_part, w2, b2, aemb32, phemb16, chemb16,
      pos50, gamma2, beta2)


def kernel(peptide_tokens, decoration, charge, node_feat, edge_index,
           node_graph_ids, pos_emb, charge_emb, a_emb, phos_emb,
           gin_W1, gin_b1, gin_W2, gin_b2, ln_gamma, ln_beta):
    f32 = jnp.float32
    p0 = _node_matmul(node_feat, gin_W1)

    src2d = edge_index[0].reshape(_NTILES, _ECH, 128).astype(jnp.int32)
    dst2d = edge_index[1].reshape(_NTILES, _ECH, 128).astype(jnp.int32)
    gid2d = node_graph_ids.reshape(_NTILES, _NCH, 128).astype(jnp.int32)

    g_part = _make_gin_sc()(p0, src2d, dst2d, gid2d, gin_b1)

    tok_flat = peptide_tokens.reshape(N * L, 1).astype(jnp.int32)
    dec_flat = decoration.reshape(N * L, 1).astype(jnp.int32)
    chg3 = charge.reshape(N // _BN, 1, _BN).astype(jnp.int32)
    aemb32 = jnp.pad(a_emb, ((0, 2), (0, 0)))
    phemb16 = jnp.pad(phos_emb, ((0, 6), (0, 0)))
    chemb16 = jnp.pad(charge_emb, ((0, 6), (0, 0)))
    pos50 = pos_emb[:L]
    out = _embed_ln(tok_flat, dec_flat, chg3, g_part, gin_W2,
                    gin_b2.reshape(1, H), aemb32, phemb16, chemb16, pos50,
                    ln_gamma.reshape(1, H), ln_beta.reshape(1, H))
    return out.reshape(N, L, H).astype(f32)


# speedup vs baseline: 7.1376x; 1.2094x over previous
import functools

import jax
import jax.numpy as jnp
from jax import lax
from jax.experimental import pallas as pl
from jax.experimental.pallas import tpu as pltpu
from jax.experimental.pallas import tpu_sc as plsc

N, L, H = 1024, 50, 128
NUM_NODES, NUM_EDGES = 20480, 81920
GIN_IN, GIN_HID = 20, 16
LN_EPS = 1e-12

_NTILES = 16
_EPT = NUM_EDGES // _NTILES
_ECH = _EPT // 128
_NPT = NUM_NODES // _NTILES
_NCH = _NPT // 128
_GPT = N // _NTILES


def _mm_body(x_ref, w_ref, o_ref):
    o_ref[...] = jnp.dot(x_ref[...], w_ref[...], preferred_element_type=jnp.float32)


def _node_matmul(node_feat, w1):
    blk = 2048
    return pl.pallas_call(
        _mm_body,
        grid=(NUM_NODES // blk,),
        in_specs=[
            pl.BlockSpec((blk, GIN_IN), lambda i: (i, 0)),
            pl.BlockSpec((GIN_IN, GIN_HID), lambda i: (0, 0)),
        ],
        out_specs=pl.BlockSpec((blk, GIN_HID), lambda i: (i, 0)),
        out_shape=jax.ShapeDtypeStruct((NUM_NODES, GIN_HID), jnp.float32),
    )(node_feat, w1)


def _gin_sc_body(p0_hbm, src_hbm, dst_hbm, gid_hbm, b1_hbm, gout_hbm,
                 src_v, dst_v, rows_v, p_v, agg_v, gid_v, g_v, b1_v,
                 agg_sh, g_sh, sem):
    c = lax.axis_index("c")
    s = lax.axis_index("s")

    def zrow(i, _):
        p_v[i, :] = jnp.zeros((16,), jnp.float32)
        return 0
    lax.fori_loop(0, _NPT, zrow, 0)
    pltpu.sync_copy(p_v, agg_sh.at[pl.ds(s * _NPT, _NPT)])
    pltpu.sync_copy(p_v.at[pl.ds(0, _GPT)], g_sh.at[pl.ds(s * _GPT, _GPT)])
    plsc.subcore_barrier()

    pltpu.sync_copy(src_hbm.at[s], src_v)
    pltpu.sync_copy(dst_hbm.at[s], dst_v)

    def echunk(j, _):
        pltpu.async_copy(p0_hbm.at[src_v.at[j]], rows_v, sem).wait()
        pltpu.sync_copy(rows_v, agg_sh.at[dst_v.at[j]], add=True)
        return 0
    lax.fori_loop(0, _ECH, echunk, 0)
    plsc.subcore_barrier()

    nbase = s * _NPT
    pltpu.sync_copy(p0_hbm.at[pl.ds(nbase, _NPT)], p_v)
    pltpu.sync_copy(agg_sh.at[pl.ds(nbase, _NPT)], agg_v)
    pltpu.sync_copy(b1_hbm, b1_v)
    b1vec = b1_v[...]

    def hrow(i, _):
        p_v[i, :] = jnp.maximum(p_v[i, :] + agg_v[i, :] + b1vec, 0.0)
        return 0
    lax.fori_loop(0, _NPT, hrow, 0)

    pltpu.sync_copy(gid_hbm.at[s], gid_v)

    def gchunk(j, _):
        pltpu.sync_copy(p_v.at[pl.ds(j * 128, 128)], g_sh.at[gid_v.at[j]], add=True)
        return 0
    lax.fori_loop(0, _NCH, gchunk, 0)
    plsc.subcore_barrier()

    pltpu.sync_copy(g_sh.at[pl.ds(s * _GPT, _GPT)], g_v)
    pltpu.sync_copy(g_v, gout_hbm.at[c, pl.ds(s * _GPT, _GPT)])


@functools.lru_cache(maxsize=1)
def _make_gin_sc():
    return functools.partial(
        pl.kernel,
        out_type=jax.ShapeDtypeStruct((2, N, GIN_HID), jnp.float32),
        mesh=plsc.VectorSubcoreMesh(core_axis_name="c", subcore_axis_name="s"),
        compiler_params=pltpu.CompilerParams(use_tc_tiling_on_sc=False),
        scratch_types=[
        pltpu.VMEM((_ECH, 128), jnp.int32),
        pltpu.VMEM((_ECH, 128), jnp.int32),
        pltpu.VMEM((128, GIN_HID), jnp.float32),
        pltpu.VMEM((_NPT, GIN_HID), jnp.float32),
        pltpu.VMEM((_NPT, GIN_HID), jnp.float32),
        pltpu.VMEM((_NCH, 128), jnp.int32),
        pltpu.VMEM((_GPT, GIN_HID), jnp.float32),
            pltpu.VMEM((GIN_HID,), jnp.float32),
            pltpu.VMEM_SHARED((NUM_NODES, GIN_HID), jnp.float32),
            pltpu.VMEM_SHARED((N, GIN_HID), jnp.float32),
            pltpu.SemaphoreType.DMA,
        ],
    )(_gin_sc_body)


_BN = 64
_BF = _BN * L


_LP = 56
_BFP = _BN * _LP


def _embed_body(tokf_ref, decf_ref, dec3_ref, chg_ref, gp_ref, w2_ref, b2_ref,
                tbl_ref, chemb_ref, pos_ref, gam_ref, bet_ref, o_ref):
    f32 = jnp.float32
    g = gp_ref[0] + gp_ref[1]
    gly = jnp.dot(g, w2_ref[...], preferred_element_type=f32) + b2_ref[...]

    chg = chg_ref[0]
    ioc = lax.broadcasted_iota(jnp.int32, (16, _BN), 0)
    onehot_c = (ioc == jnp.broadcast_to(chg, (16, _BN))).astype(f32)
    chg_emb = lax.dot_general(onehot_c, chemb_ref[...],
                              (((0,), (0,)), ((), ())),
                              preferred_element_type=f32)

    tok = tokf_ref[...]
    dec = decf_ref[...]
    j = lax.broadcasted_iota(jnp.int32, (_BFP, 48), 1)
    m2 = ((j == tok).astype(f32) + ((j - 32) == dec).astype(f32))
    emb2 = jnp.dot(m2, tbl_ref[...], preferred_element_type=f32)

    emb3 = emb2.reshape(_BN, _LP, H)
    emb3 = emb3 + chg_emb[:, None, :] + pos_ref[...][None, :, :]
    mask3 = (dec3_ref[...] == 5).astype(f32)[:, :, None]
    emb3 = emb3 + mask3 * gly[:, None, :]

    mu = jnp.mean(emb3, axis=2, keepdims=True)
    d = emb3 - mu
    var = jnp.mean(d * d, axis=2, keepdims=True)
    res = d * lax.rsqrt(var + LN_EPS)
    res = res * gam_ref[...][None, :, :] + bet_ref[...][None, :, :]
    o_ref[...] = res[:, :L, :]


def _embed_ln(tokf, decf, dec3, chg3, g_part, w2, b2, tbl48, chemb16,
              pos56, gamma2, beta2):
    nblk = N // _BN
    return pl.pallas_call(
        _embed_body,
        grid=(nblk,),
        in_specs=[
            pl.BlockSpec((_BFP, 1), lambda i: (i, 0)),
            pl.BlockSpec((_BFP, 1), lambda i: (i, 0)),
            pl.BlockSpec((_BN, _LP), lambda i: (i, 0)),
            pl.BlockSpec((1, 1, _BN), lambda i: (i, 0, 0)),
            pl.BlockSpec((2, _BN, GIN_HID), lambda i: (0, i, 0)),
            pl.BlockSpec((GIN_HID, H), lambda i: (0, 0)),
            pl.BlockSpec((1, H), lambda i: (0, 0)),
            pl.BlockSpec((48, H), lambda i: (0, 0)),
            pl.BlockSpec((16, H), lambda i: (0, 0)),
            pl.BlockSpec((_LP, H), lambda i: (0, 0)),
            pl.BlockSpec((1, H), lambda i: (0, 0)),
            pl.BlockSpec((1, H), lambda i: (0, 0)),
        ],
        out_specs=pl.BlockSpec((_BN, L, H), lambda i: (i, 0, 0)),
        out_shape=jax.ShapeDtypeStruct((N, L, H), jnp.float32),
    )(tokf, decf, dec3, chg3, g_part, w2, b2, tbl48, chemb16, pos56,
      gamma2, beta2)


def kernel(peptide_tokens, decoration, charge, node_feat, edge_index,
           node_graph_ids, pos_emb, charge_emb, a_emb, phos_emb,
           gin_W1, gin_b1, gin_W2, gin_b2, ln_gamma, ln_beta):
    f32 = jnp.float32
    p0 = _node_matmul(node_feat, gin_W1)

    src2d = edge_index[0].reshape(_NTILES, _ECH, 128).astype(jnp.int32)
    dst2d = edge_index[1].reshape(_NTILES, _ECH, 128).astype(jnp.int32)
    gid2d = node_graph_ids.reshape(_NTILES, _NCH, 128).astype(jnp.int32)

    g_part = _make_gin_sc()(p0, src2d, dst2d, gid2d, gin_b1)

    tok_p = jnp.pad(peptide_tokens.astype(jnp.int32), ((0, 0), (0, _LP - L)))
    dec_p = jnp.pad(decoration.astype(jnp.int32), ((0, 0), (0, _LP - L)),
                    constant_values=-1)
    tokf = tok_p.reshape(N * _LP, 1)
    decf = dec_p.reshape(N * _LP, 1)
    chg3 = charge.reshape(N // _BN, 1, _BN).astype(jnp.int32)
    tbl48 = jnp.concatenate(
        [a_emb, jnp.zeros((2, H), f32), phos_emb, jnp.zeros((6, H), f32)], axis=0)
    chemb16 = jnp.pad(charge_emb, ((0, 6), (0, 0)))
    pos56 = jnp.pad(pos_emb[:L], ((0, _LP - L), (0, 0)))
    out = _embed_ln(tokf, decf, dec_p, chg3, g_part, gin_W2,
                    gin_b2.reshape(1, H), tbl48, chemb16, pos56,
                    ln_gamma.reshape(1, H), ln_beta.reshape(1, H))
    return out.astype(f32)


# speedup vs baseline: 8.4044x; 1.1775x over previous
import functools

import jax
import jax.numpy as jnp
from jax import lax
from jax.experimental import pallas as pl
from jax.experimental.pallas import tpu as pltpu
from jax.experimental.pallas import tpu_sc as plsc

N, L, H = 1024, 50, 128
NUM_NODES, NUM_EDGES = 20480, 81920
GIN_IN, GIN_HID = 20, 16
LN_EPS = 1e-12

_NTILES = 16
_EPT = NUM_EDGES // _NTILES
_ECH = _EPT // 128
_NPT = NUM_NODES // _NTILES
_NCH = _NPT // 128
_GPT = N // _NTILES


def _mm_body(x_ref, w_ref, o_ref):
    o_ref[...] = jnp.dot(x_ref[...], w_ref[...], preferred_element_type=jnp.float32)


def _node_matmul(node_feat, w1):
    blk = 2048
    return pl.pallas_call(
        _mm_body,
        grid=(NUM_NODES // blk,),
        in_specs=[
            pl.BlockSpec((blk, GIN_IN), lambda i: (i, 0)),
            pl.BlockSpec((GIN_IN, GIN_HID), lambda i: (0, 0)),
        ],
        out_specs=pl.BlockSpec((blk, GIN_HID), lambda i: (i, 0)),
        out_shape=jax.ShapeDtypeStruct((NUM_NODES, GIN_HID), jnp.float32),
    )(node_feat, w1)


def _gin_sc_body(p0_hbm, src_hbm, dst_hbm, gid_hbm, b1_hbm, gout_hbm,
                 src_v, dst_v, rows_a, rows_b, p_v, agg_v, gid_v, b1_v,
                 agg_sh, g_sh, sem_a, sem_b, sem_p):
    c = lax.axis_index("c")
    s = lax.axis_index("s")
    nbase = s * _NPT

    pltpu.async_copy(p0_hbm.at[pl.ds(nbase, _NPT)], p_v, sem_p)
    pltpu.sync_copy(src_hbm.at[s], src_v)
    pltpu.sync_copy(dst_hbm.at[s], dst_v)
    pltpu.sync_copy(gid_hbm.at[s], gid_v)
    pltpu.sync_copy(b1_hbm, b1_v)

    def zrow(i, _):
        rows_a[i, :] = jnp.zeros((16,), jnp.float32)
        return 0
    lax.fori_loop(0, 128, zrow, 0)

    def zcp(k, _):
        pltpu.sync_copy(rows_a, agg_sh.at[pl.ds(nbase + k * 128, 128)])
        return 0
    lax.fori_loop(0, _NCH, zcp, 0)
    pltpu.sync_copy(rows_a.at[pl.ds(0, _GPT)], g_sh.at[pl.ds(s * _GPT, _GPT)])
    plsc.subcore_barrier()

    pltpu.async_copy(p0_hbm.at[src_v.at[0]], rows_a, sem_a)

    def epair(jj, _):
        j0 = 2 * jj
        j1 = j0 + 1
        pltpu.async_copy(p0_hbm.at[src_v.at[j1]], rows_b, sem_b)
        pltpu.make_async_copy(p0_hbm.at[src_v.at[j0]], rows_a, sem_a).wait()
        pltpu.sync_copy(rows_a, agg_sh.at[dst_v.at[j0]], add=True)

        @pl.when(jj < _ECH // 2 - 1)
        def _():
            pltpu.async_copy(p0_hbm.at[src_v.at[j0 + 2]], rows_a, sem_a)

        pltpu.make_async_copy(p0_hbm.at[src_v.at[j1]], rows_b, sem_b).wait()
        pltpu.sync_copy(rows_b, agg_sh.at[dst_v.at[j1]], add=True)
        return 0
    lax.fori_loop(0, _ECH // 2, epair, 0)
    plsc.subcore_barrier()

    pltpu.make_async_copy(p0_hbm.at[pl.ds(nbase, _NPT)], p_v, sem_p).wait()
    pltpu.sync_copy(agg_sh.at[pl.ds(nbase, _NPT)], agg_v)
    b1vec = b1_v[...]

    def hrow8(i8, _):
        for k in range(8):
            i = i8 * 8 + k
            p_v[i, :] = jnp.maximum(p_v[i, :] + agg_v[i, :] + b1vec, 0.0)
        return 0
    lax.fori_loop(0, _NPT // 8, hrow8, 0)

    def gchunk(j, _):
        pltpu.sync_copy(p_v.at[pl.ds(j * 128, 128)], g_sh.at[gid_v.at[j]], add=True)
        return 0
    lax.fori_loop(0, _NCH, gchunk, 0)
    plsc.subcore_barrier()

    pltpu.sync_copy(g_sh.at[pl.ds(s * _GPT, _GPT)], gout_hbm.at[c, pl.ds(s * _GPT, _GPT)])


@functools.lru_cache(maxsize=1)
def _make_gin_sc():
    return functools.partial(
        pl.kernel,
        out_type=jax.ShapeDtypeStruct((2, N, GIN_HID), jnp.float32),
        mesh=plsc.VectorSubcoreMesh(core_axis_name="c", subcore_axis_name="s"),
        compiler_params=pltpu.CompilerParams(use_tc_tiling_on_sc=False),
        scratch_types=[
            pltpu.VMEM((_ECH, 128), jnp.int32),
            pltpu.VMEM((_ECH, 128), jnp.int32),
            pltpu.VMEM((128, GIN_HID), jnp.float32),
            pltpu.VMEM((128, GIN_HID), jnp.float32),
            pltpu.VMEM((_NPT, GIN_HID), jnp.float32),
            pltpu.VMEM((_NPT, GIN_HID), jnp.float32),
            pltpu.VMEM((_NCH, 128), jnp.int32),
            pltpu.VMEM((GIN_HID,), jnp.float32),
            pltpu.VMEM_SHARED((NUM_NODES, GIN_HID), jnp.float32),
            pltpu.VMEM_SHARED((N, GIN_HID), jnp.float32),
            pltpu.SemaphoreType.DMA,
            pltpu.SemaphoreType.DMA,
            pltpu.SemaphoreType.DMA,
        ],
    )(_gin_sc_body)


_BN = 64
_BF = _BN * L


_LP = 56
_BFP = _BN * _LP


def _embed_body(tokf_ref, decf_ref, dec3_ref, chg_ref, gp_ref, w2_ref, b2_ref,
                tbl_ref, chemb_ref, pos_ref, gam_ref, bet_ref, o_ref):
    f32 = jnp.float32
    g = gp_ref[0] + gp_ref[1]
    gly = jnp.dot(g, w2_ref[...], preferred_element_type=f32) + b2_ref[...]

    chg = chg_ref[0]
    ioc = lax.broadcasted_iota(jnp.int32, (16, _BN), 0)
    onehot_c = (ioc == jnp.broadcast_to(chg, (16, _BN))).astype(f32)
    chg_emb = lax.dot_general(onehot_c, chemb_ref[...],
                              (((0,), (0,)), ((), ())),
                              preferred_element_type=f32)

    tok = tokf_ref[...]
    dec = decf_ref[...]
    j = lax.broadcasted_iota(jnp.int32, (_BFP, 48), 1)
    m2 = ((j == tok).astype(f32) + ((j - 32) == dec).astype(f32))
    emb2 = jnp.dot(m2, tbl_ref[...], preferred_element_type=f32)

    emb3 = emb2.reshape(_BN, _LP, H)
    emb3 = emb3 + chg_emb[:, None, :] + pos_ref[...][None, :, :]
    mask3 = (dec3_ref[...] == 5).astype(f32)[:, :, None]
    emb3 = emb3 + mask3 * gly[:, None, :]

    mu = jnp.mean(emb3, axis=2, keepdims=True)
    d = emb3 - mu
    var = jnp.mean(d * d, axis=2, keepdims=True)
    res = d * lax.rsqrt(var + LN_EPS)
    res = res * gam_ref[...][None, :, :] + bet_ref[...][None, :, :]
    o_ref[...] = res[:, :L, :]


def _embed_ln(tokf, decf, dec3, chg3, g_part, w2, b2, tbl48, chemb16,
              pos56, gamma2, beta2):
    nblk = N // _BN
    return pl.pallas_call(
        _embed_body,
        grid=(nblk,),
        in_specs=[
            pl.BlockSpec((_BFP, 1), lambda i: (i, 0)),
            pl.BlockSpec((_BFP, 1), lambda i: (i, 0)),
            pl.BlockSpec((_BN, _LP), lambda i: (i, 0)),
            pl.BlockSpec((1, 1, _BN), lambda i: (i, 0, 0)),
            pl.BlockSpec((2, _BN, GIN_HID), lambda i: (0, i, 0)),
            pl.BlockSpec((GIN_HID, H), lambda i: (0, 0)),
            pl.BlockSpec((1, H), lambda i: (0, 0)),
            pl.BlockSpec((48, H), lambda i: (0, 0)),
            pl.BlockSpec((16, H), lambda i: (0, 0)),
            pl.BlockSpec((_LP, H), lambda i: (0, 0)),
            pl.BlockSpec((1, H), lambda i: (0, 0)),
            pl.BlockSpec((1, H), lambda i: (0, 0)),
        ],
        out_specs=pl.BlockSpec((_BN, L, H), lambda i: (i, 0, 0)),
        out_shape=jax.ShapeDtypeStruct((N, L, H), jnp.float32),
    )(tokf, decf, dec3, chg3, g_part, w2, b2, tbl48, chemb16, pos56,
      gamma2, beta2)


def kernel(peptide_tokens, decoration, charge, node_feat, edge_index,
           node_graph_ids, pos_emb, charge_emb, a_emb, phos_emb,
           gin_W1, gin_b1, gin_W2, gin_b2, ln_gamma, ln_beta):
    f32 = jnp.float32
    p0 = _node_matmul(node_feat, gin_W1)

    src2d = edge_index[0].reshape(_NTILES, _ECH, 128).astype(jnp.int32)
    dst2d = edge_index[1].reshape(_NTILES, _ECH, 128).astype(jnp.int32)
    gid2d = node_graph_ids.reshape(_NTILES, _NCH, 128).astype(jnp.int32)

    g_part = _make_gin_sc()(p0, src2d, dst2d, gid2d, gin_b1)

    tok_p = jnp.pad(peptide_tokens.astype(jnp.int32), ((0, 0), (0, _LP - L)))
    dec_p = jnp.pad(decoration.astype(jnp.int32), ((0, 0), (0, _LP - L)),
                    constant_values=-1)
    tokf = tok_p.reshape(N * _LP, 1)
    decf = dec_p.reshape(N * _LP, 1)
    chg3 = charge.reshape(N // _BN, 1, _BN).astype(jnp.int32)
    tbl48 = jnp.concatenate(
        [a_emb, jnp.zeros((2, H), f32), phos_emb, jnp.zeros((6, H), f32)], axis=0)
    chemb16 = jnp.pad(charge_emb, ((0, 6), (0, 0)))
    pos56 = jnp.pad(pos_emb[:L], ((0, _LP - L), (0, 0)))
    out = _embed_ln(tokf, decf, dec_p, chg3, g_part, gin_W2,
                    gin_b2.reshape(1, H), tbl48, chemb16, pos56,
                    ln_gamma.reshape(1, H), ln_beta.reshape(1, H))
    return out.astype(f32)
